# baseline (speedup 1.0000x reference)
import functools

import jax
import jax.numpy as jnp
from jax import lax
from jax.experimental import pallas as pl
from jax.experimental.pallas import tpu as pltpu
from jax.experimental.pallas import tpu_sc as plsc

D = 16
N_NODES = 10000
N_EDGES = 160000

NC = 2
NS = 16
NW = NC * NS
CHUNK = 128
EPW = 5120
NCHUNK = EPW // CHUNK
E_PAD = NW * EPW
N_PAD = 10240
NROW128 = N_PAD * D // 128
TPT = NROW128 // NS
EROW128 = E_PAD * D // 128
EPT = EROW128 // NW
EDGE_BLOCK = 4096

_SC_MESH = plsc.VectorSubcoreMesh(core_axis_name="c", subcore_axis_name="s")


def _gather_body(tab_hbm, src2d_hbm, out_hbm, idx_v, rows_v, tab_sh, sem):
    c = lax.axis_index("c")
    s = lax.axis_index("s")
    wid = s * NC + c
    base = wid * NCHUNK
    npt = N_PAD // NS
    pltpu.sync_copy(tab_hbm.at[pl.ds(s * npt, npt)],
                    tab_sh.at[pl.ds(s * npt, npt)])
    pltpu.sync_copy(src2d_hbm.at[pl.ds(base, NCHUNK)], idx_v)
    plsc.subcore_barrier()

    def fire(j, carry):
        pltpu.async_copy(tab_sh.at[idx_v.at[j]],
                         rows_v.at[pl.ds(j * CHUNK, CHUNK)], sem)
        return carry

    lax.fori_loop(0, NCHUNK, fire, 0)

    def drain(j, carry):
        pltpu.make_async_copy(tab_sh.at[idx_v.at[j]],
                              rows_v.at[pl.ds(j * CHUNK, CHUNK)],
                              sem).wait()
        return carry

    lax.fori_loop(0, NCHUNK, drain, 0)
    pltpu.sync_copy(rows_v, out_hbm.at[pl.ds(wid * EPW, EPW)])


def _sc_gather(tab, src2d):
    return pl.kernel(
        _gather_body,
        out_type=jax.ShapeDtypeStruct((E_PAD, D), jnp.float32),
        mesh=_SC_MESH,
        compiler_params=pltpu.CompilerParams(use_tc_tiling_on_sc=False),
        scratch_types=[
            pltpu.VMEM((NCHUNK, CHUNK), jnp.int32),
            pltpu.VMEM((EPW, D), jnp.float32),
            pltpu.VMEM_SHARED((N_PAD, D), jnp.float32),
            pltpu.SemaphoreType.DMA,
        ],
    )(tab, src2d)


def _scatter_body(msg_hbm, dst2d_hbm, zeros_hbm, out_hbm, idx_v, rows_v,
                  obuf_v, agg_sh, sem):
    c = lax.axis_index("c")
    s = lax.axis_index("s")
    wid = s * NC + c
    base = wid * NCHUNK
    npt = N_PAD // NS
    pltpu.sync_copy(dst2d_hbm.at[pl.ds(base, NCHUNK)], idx_v)
    pltpu.async_copy(msg_hbm.at[pl.ds(wid * EPW, EPW)], rows_v, sem)
    pltpu.sync_copy(zeros_hbm.at[pl.ds(s * npt, npt)],
                    agg_sh.at[pl.ds(s * npt, npt)])
    pltpu.make_async_copy(msg_hbm.at[pl.ds(wid * EPW, EPW)], rows_v,
                          sem).wait()
    plsc.subcore_barrier()

    def body(j, carry):
        pltpu.sync_copy(rows_v.at[pl.ds(j * CHUNK, CHUNK)],
                        agg_sh.at[idx_v.at[j]], add=True)
        return carry

    lax.fori_loop(0, NCHUNK, body, 0)
    plsc.subcore_barrier()
    pltpu.sync_copy(agg_sh.at[pl.ds(s * npt, npt)], obuf_v)
    pltpu.sync_copy(obuf_v, out_hbm.at[c, pl.ds(s * npt, npt)])


def _sc_scatter_add(msg, dst2d, zeros_nd):
    return pl.kernel(
        _scatter_body,
        out_type=jax.ShapeDtypeStruct((NC, N_PAD, D), jnp.float32),
        mesh=_SC_MESH,
        compiler_params=pltpu.CompilerParams(use_tc_tiling_on_sc=False),
        scratch_types=[
            pltpu.VMEM((NCHUNK, CHUNK), jnp.int32),
            pltpu.VMEM((EPW, D), jnp.float32),
            pltpu.VMEM((N_PAD // NS, D), jnp.float32),
            pltpu.VMEM_SHARED((N_PAD, D), jnp.float32),
            pltpu.SemaphoreType.DMA,
        ],
    )(msg, dst2d, zeros_nd)


def _embed_body(x_ref, w0t_ref, b0_ref, o_ref):
    o_ref[...] = jnp.maximum(
        jnp.dot(x_ref[...], w0t_ref[...], preferred_element_type=jnp.float32)
        + b0_ref[...], 0.0)


def _node_embed(xp, W0, b0):
    return pl.pallas_call(
        _embed_body,
        out_shape=jax.ShapeDtypeStruct((N_PAD, D), jnp.float32),
    )(xp, W0.T, b0.reshape(1, D))


def _u_body(ea_ref, w1row_ref, be1_ref, o_ref):
    o_ref[...] = jnp.maximum(
        ea_ref[...] * w1row_ref[...] + be1_ref[...], 0.0)


def _u_table(ea_pad, w1row, be1):
    return pl.pallas_call(
        _u_body,
        grid=(20,),
        in_specs=[
            pl.BlockSpec((E_PAD // 20, 1), lambda i: (i, 0)),
            pl.BlockSpec((1, D), lambda i: (0, 0)),
            pl.BlockSpec((1, D), lambda i: (0, 0)),
        ],
        out_specs=pl.BlockSpec((E_PAD // 20, D), lambda i: (i, 0)),
        out_shape=jax.ShapeDtypeStruct((E_PAD, D), jnp.float32),
    )(ea_pad, w1row, be1)


MSG_ROWS = 2048


def _msg_body(xp_ref, up_ref, M_ref, Bm_ref, o_ref):
    X = xp_ref[...]
    U = up_ref[...]
    M = M_ref[...]
    Bm = Bm_ref[...]
    parts = []
    for g in range(8):
        xg = X[:, D * g:D * (g + 1)]
        ug = U[:, D * g:D * (g + 1)]
        Pg = (xg[:, :, None] * ug[:, None, :]).reshape(MSG_ROWS, D * D)
        parts.append(jnp.dot(Pg, M, preferred_element_type=jnp.float32)
                     + jnp.dot(xg, Bm, preferred_element_type=jnp.float32))
    o_ref[...] = jnp.concatenate(parts, axis=1)


def _messages(xjp, up, M, Bm):
    grid = EROW128 // MSG_ROWS
    return pl.pallas_call(
        _msg_body,
        grid=(grid,),
        in_specs=[
            pl.BlockSpec((MSG_ROWS, 128), lambda i: (i, 0)),
            pl.BlockSpec((MSG_ROWS, 128), lambda i: (i, 0)),
            pl.BlockSpec((D * D, D), lambda i: (0, 0)),
            pl.BlockSpec((D, D), lambda i: (0, 0)),
        ],
        out_specs=pl.BlockSpec((MSG_ROWS, 128), lambda i: (i, 0)),
        out_shape=jax.ShapeDtypeStruct((EROW128, 128), jnp.float32),
    )(xjp, up, M, Bm)


def _gru_body(agg0_ref, agg1_ref, deg0_ref, deg1_ref, out_ref, h_ref,
              wroott_ref, bconv_ref, wih_ref, bih_ref, whh_ref, bhh_ref,
              newh_ref):
    parts = []
    for g in range(8):
        sl = slice(D * g, D * (g + 1))
        out = out_ref[:, sl]
        h = h_ref[:, sl]
        invdeg = 1.0 / jnp.maximum(deg0_ref[:, sl] + deg1_ref[:, sl], 1.0)
        agg = (agg0_ref[:, sl] + agg1_ref[:, sl]) * invdeg
        m = jnp.maximum(
            agg + jnp.dot(out, wroott_ref[...],
                          preferred_element_type=jnp.float32)
            + bconv_ref[...], 0.0)
        gi = (jnp.dot(m, wih_ref[...], preferred_element_type=jnp.float32)
              + bih_ref[...])
        gh = (jnp.dot(h, whh_ref[...], preferred_element_type=jnp.float32)
              + bhh_ref[...])
        r = jax.nn.sigmoid(gi[:, :D] + gh[:, :D])
        z = jax.nn.sigmoid(gi[:, D:2 * D] + gh[:, D:2 * D])
        n = jnp.tanh(gi[:, 2 * D:] + r * gh[:, 2 * D:])
        parts.append((1.0 - z) * n + z * h)
    newh_ref[...] = jnp.concatenate(parts, axis=1)


def _gru(agg0, agg1, deg0, deg1, outp, hp, WrootT, bconv, WihT, bih, WhhT,
         bhh):
    return pl.pallas_call(
        _gru_body,
        out_shape=jax.ShapeDtypeStruct((NROW128, 128), jnp.float32),
    )(agg0, agg1, deg0, deg1, outp, hp, WrootT, bconv, WihT, bih, WhhT, bhh)


def _set2set_body(out_ref, wihs_ref, bihs_ref, whhs_ref, bhhs_ref,
                  wihm_ref, bm_ref, hx_ref, cx_ref):
    out = out_ref[...]
    q_star = jnp.zeros((1, 2 * D), jnp.float32)
    hs = jnp.zeros((1, D), jnp.float32)
    cs = jnp.zeros((1, D), jnp.float32)
    for _ in range(6):
        g = (jnp.dot(q_star, wihs_ref[...], preferred_element_type=jnp.float32)
             + bihs_ref[...]
             + jnp.dot(hs, whhs_ref[...], preferred_element_type=jnp.float32)
             + bhhs_ref[...])
        ig = jax.nn.sigmoid(g[:, :D])
        fg = jax.nn.sigmoid(g[:, D:2 * D])
        cg = jnp.tanh(g[:, 2 * D:3 * D])
        og = jax.nn.sigmoid(g[:, 3 * D:])
        cs = fg * cs + ig * cg
        hs = og * jnp.tanh(cs)
        e = jnp.sum(out * hs, axis=1, keepdims=True)
        emax = jnp.max(e)
        a = jnp.exp(e - emax)
        asum = jnp.sum(a)
        rvec = jnp.sum(a * out, axis=0, keepdims=True) / asum
        q_star = jnp.concatenate([hs, rvec], axis=1)
    g = (jnp.dot(q_star, wihm_ref[...], preferred_element_type=jnp.float32)
         + bm_ref[...])
    ig = jax.nn.sigmoid(g[:, :D])
    fg = jax.nn.sigmoid(g[:, D:2 * D])
    cg = jnp.tanh(g[:, 2 * D:3 * D])
    og = jax.nn.sigmoid(g[:, 3 * D:])
    cx = ig * cg
    hx_ref[...] = og * jnp.tanh(cx)
    cx_ref[...] = cx


def _set2set(out, Wih_s, bih_s, Whh_s, bhh_s, Wih_m, bih_m, bhh_m):
    return pl.pallas_call(
        _set2set_body,
        out_shape=(jax.ShapeDtypeStruct((1, D), jnp.float32),
                   jax.ShapeDtypeStruct((1, D), jnp.float32)),
    )(out, Wih_s.T, bih_s.reshape(1, -1), Whh_s.T, bhh_s.reshape(1, -1),
      Wih_m.T, (bih_m + bhh_m).reshape(1, -1))


def _mlp_body(sel_ref, repcol_ref, w1at_ref, w1bsum_ref, b1_ref, w2t_ref,
              b2_ref, o_ref):
    z2 = jnp.maximum(
        jnp.dot(sel_ref[...], w1at_ref[...], preferred_element_type=jnp.float32)
        + repcol_ref[...] * w1bsum_ref[...]
        + b1_ref[...], 0.0)
    o_ref[...] = (jnp.dot(z2, w2t_ref[...], preferred_element_type=jnp.float32)
                  + b2_ref[...])


def _mlp(sel, repcol, W1, b1, W2, b2):
    n_t = sel.shape[0]
    return pl.pallas_call(
        _mlp_body,
        out_shape=jax.ShapeDtypeStruct((n_t, W2.shape[0]), jnp.float32),
    )(sel, repcol, W1[:, :4 * D].T, W1[:, 4 * D:].sum(axis=1).reshape(1, D),
      b1.reshape(1, -1), W2.T, b2.reshape(1, -1))


def kernel(x, edge_attr, edge_index, batch, nonring, W0, b0, We1, be1, We2,
           be2, Wroot, bconv, Wih, Whh, bih, bhh, Wih_s, Whh_s, bih_s, bhh_s,
           Wih_m, Whh_m, bih_m, bhh_m, W1, b1, W2, b2):
    src = edge_index[0]
    dst = edge_index[1]

    M = We2.reshape(D, D, D).transpose(0, 2, 1).reshape(D * D, D)
    Bm = be2.reshape(D, D)
    w1row = We1.T
    be1r = be1.reshape(1, D)

    n_fill = E_PAD - N_EDGES
    fill = jnp.arange(n_fill, dtype=jnp.int32)
    src_pad = jnp.concatenate([src, (fill * 521) % N_NODES])
    dst_pad = jnp.concatenate([dst, N_NODES + (fill % (N_PAD - N_NODES))])
    src2d = src_pad.reshape(E_PAD // CHUNK, CHUNK)
    dst2d = dst_pad.reshape(E_PAD // CHUNK, CHUNK)
    ea_pad = jnp.concatenate(
        [edge_attr, jnp.zeros((n_fill, 1), jnp.float32)])
    zeros_nd = jnp.zeros((N_PAD, D), jnp.float32)
    xp = jnp.concatenate([x, jnp.zeros((N_PAD - N_NODES, x.shape[1]),
                                       jnp.float32)])

    out16 = _node_embed(xp, W0, b0)
    outp = out16.reshape(NROW128, 128)
    hp = outp

    u16 = _u_table(ea_pad, w1row, be1r)
    up = u16.reshape(EROW128, 128)

    ones_nd = jnp.ones((E_PAD, D), jnp.float32)
    degp = _sc_scatter_add(ones_nd, dst2d, zeros_nd)
    degpk = degp.reshape(NC, NROW128, 128)
    deg0 = degpk[0]
    deg1 = degpk[1]

    WrootT = Wroot.T
    bconvr = bconv.reshape(1, D)
    WihT = Wih.T
    bihr = bih.reshape(1, 3 * D)
    WhhT = Whh.T
    bhhr = bhh.reshape(1, 3 * D)

    for _ in range(6):
        xj = _sc_gather(outp.reshape(N_PAD, D), src2d)
        msgp = _messages(xj.reshape(EROW128, 128), up, M, Bm)
        aggp = _sc_scatter_add(msgp.reshape(E_PAD, D), dst2d, zeros_nd)
        aggpk = aggp.reshape(NC, NROW128, 128)
        hp = _gru(aggpk[0], aggpk[1], deg0, deg1, outp, hp,
                  WrootT, bconvr, WihT, bihr, WhhT, bhhr)
        outp = hp

    out = outp.reshape(N_PAD, D)
    outv = out[:N_NODES]
    hx, cx = _set2set(outv, Wih_s, bih_s, Whh_s, bhh_s, Wih_m, bih_m, bhh_m)

    sel_rows = jnp.take(outv, nonring.reshape(-1), axis=0)
    sel = sel_rows.reshape(4 * D, -1).T
    n_t = sel.shape[0]
    repcol = jnp.repeat(hx.reshape(-1), n_t // D).reshape(n_t, 1)
    logits = _mlp(sel, repcol, W1, b1, W2, b2)
    return logits, hx, cx


# speedup vs baseline: 4.5215x; 4.5215x over previous
import functools

import jax
import jax.numpy as jnp
from jax import lax
from jax.experimental import pallas as pl
from jax.experimental.pallas import tpu as pltpu
from jax.experimental.pallas import tpu_sc as plsc

D = 16
N_NODES = 10000
N_EDGES = 160000

NC = 2
NS = 16
NW = NC * NS
CHUNK = 128
EPW = 5120
NCHUNK = EPW // CHUNK
E_PAD = NW * EPW
N_PAD = 10240
NROW128 = N_PAD * D // 128
TPT = NROW128 // NS
EROW128 = E_PAD * D // 128
EPT = EROW128 // NW
EDGE_BLOCK = 4096

_SC_MESH = plsc.VectorSubcoreMesh(core_axis_name="c", subcore_axis_name="s")


def _gather_body(tab_hbm, src2d_hbm, out_hbm, idx_v, rows_v, tab_sh, sem):
    c = lax.axis_index("c")
    s = lax.axis_index("s")
    wid = s * NC + c
    base = wid * NCHUNK
    npt = N_PAD // NS
    pltpu.sync_copy(tab_hbm.at[pl.ds(s * npt, npt)],
                    tab_sh.at[pl.ds(s * npt, npt)])
    pltpu.sync_copy(src2d_hbm.at[pl.ds(base, NCHUNK)], idx_v)
    plsc.subcore_barrier()

    def fire(j, carry):
        pltpu.async_copy(tab_sh.at[idx_v.at[j]],
                         rows_v.at[pl.ds(j * CHUNK, CHUNK)], sem)
        return carry

    lax.fori_loop(0, NCHUNK, fire, 0)

    def drain(j, carry):
        pltpu.make_async_copy(tab_sh.at[idx_v.at[j]],
                              rows_v.at[pl.ds(j * CHUNK, CHUNK)],
                              sem).wait()
        return carry

    lax.fori_loop(0, NCHUNK, drain, 0)
    pltpu.sync_copy(rows_v, out_hbm.at[pl.ds(wid * EPW, EPW)])


def _sc_gather(tab, src2d):
    return pl.kernel(
        _gather_body,
        out_type=jax.ShapeDtypeStruct((E_PAD, D), jnp.float32),
        mesh=_SC_MESH,
        compiler_params=pltpu.CompilerParams(use_tc_tiling_on_sc=False),
        scratch_types=[
            pltpu.VMEM((NCHUNK, CHUNK), jnp.int32),
            pltpu.VMEM((EPW, D), jnp.float32),
            pltpu.VMEM_SHARED((N_PAD, D), jnp.float32),
            pltpu.SemaphoreType.DMA,
        ],
    )(tab, src2d)


def _scatter_body(msg_hbm, dst2d_hbm, zeros_hbm, out_hbm, idx_v, rows_v,
                  obuf_v, agg_sh, sem):
    c = lax.axis_index("c")
    s = lax.axis_index("s")
    wid = s * NC + c
    base = wid * NCHUNK
    npt = N_PAD // NS
    pltpu.sync_copy(dst2d_hbm.at[pl.ds(base, NCHUNK)], idx_v)
    pltpu.async_copy(msg_hbm.at[pl.ds(wid * EPW, EPW)], rows_v, sem)
    pltpu.sync_copy(zeros_hbm.at[pl.ds(s * npt, npt)],
                    agg_sh.at[pl.ds(s * npt, npt)])
    pltpu.make_async_copy(msg_hbm.at[pl.ds(wid * EPW, EPW)], rows_v,
                          sem).wait()
    plsc.subcore_barrier()

    def body(j, carry):
        pltpu.sync_copy(rows_v.at[pl.ds(j * CHUNK, CHUNK)],
                        agg_sh.at[idx_v.at[j]], add=True)
        return carry

    lax.fori_loop(0, NCHUNK, body, 0)
    plsc.subcore_barrier()
    pltpu.sync_copy(agg_sh.at[pl.ds(s * npt, npt)], obuf_v)
    pltpu.sync_copy(obuf_v, out_hbm.at[c, pl.ds(s * npt, npt)])


def _sc_scatter_add(msg, dst2d, zeros_nd):
    return pl.kernel(
        _scatter_body,
        out_type=jax.ShapeDtypeStruct((NC, N_PAD, D), jnp.float32),
        mesh=_SC_MESH,
        compiler_params=pltpu.CompilerParams(use_tc_tiling_on_sc=False),
        scratch_types=[
            pltpu.VMEM((NCHUNK, CHUNK), jnp.int32),
            pltpu.VMEM((EPW, D), jnp.float32),
            pltpu.VMEM((N_PAD // NS, D), jnp.float32),
            pltpu.VMEM_SHARED((N_PAD, D), jnp.float32),
            pltpu.SemaphoreType.DMA,
        ],
    )(msg, dst2d, zeros_nd)


def _embed_body(x_ref, w0t_ref, b0_ref, o_ref):
    o_ref[...] = jnp.maximum(
        jnp.dot(x_ref[...], w0t_ref[...], preferred_element_type=jnp.float32)
        + b0_ref[...], 0.0)


def _node_embed(xp, W0, b0):
    return pl.pallas_call(
        _embed_body,
        out_shape=jax.ShapeDtypeStruct((N_PAD, D), jnp.float32),
    )(xp, W0.T, b0.reshape(1, D))


def _u_body(ea_ref, w1row_ref, be1_ref, o_ref):
    o_ref[...] = jnp.maximum(
        ea_ref[...] * w1row_ref[...] + be1_ref[...], 0.0)


def _u_table(ea_pad, w1row, be1):
    return pl.pallas_call(
        _u_body,
        grid=(20,),
        in_specs=[
            pl.BlockSpec((E_PAD // 20, 1), lambda i: (i, 0)),
            pl.BlockSpec((1, D), lambda i: (0, 0)),
            pl.BlockSpec((1, D), lambda i: (0, 0)),
        ],
        out_specs=pl.BlockSpec((E_PAD // 20, D), lambda i: (i, 0)),
        out_shape=jax.ShapeDtypeStruct((E_PAD, D), jnp.float32),
    )(ea_pad, w1row, be1)


MSG_ROWS = 2048


def _msg_body(xp_ref, up_ref, RA_ref, RB_ref, M_ref, Bm_ref, o_ref):
    X = xp_ref[...]
    U = up_ref[...]
    RA = RA_ref[...]
    RB = RB_ref[...]
    M = M_ref[...]
    Bm = Bm_ref[...]
    parts = []
    for g in range(8):
        xg = X[:, D * g:D * (g + 1)]
        ug = U[:, D * g:D * (g + 1)]
        A = jnp.dot(xg, RA, preferred_element_type=jnp.float32)
        B = jnp.dot(ug, RB, preferred_element_type=jnp.float32)
        parts.append(
            jnp.dot(A * B, M, preferred_element_type=jnp.float32)
            + jnp.dot(xg, Bm, preferred_element_type=jnp.float32))
    o_ref[...] = jnp.concatenate(parts, axis=1)


def _messages(xjp, up, RA, RB, M, Bm):
    grid = EROW128 // MSG_ROWS
    return pl.pallas_call(
        _msg_body,
        grid=(grid,),
        in_specs=[
            pl.BlockSpec((MSG_ROWS, 128), lambda i: (i, 0)),
            pl.BlockSpec((MSG_ROWS, 128), lambda i: (i, 0)),
            pl.BlockSpec((D, D * D), lambda i: (0, 0)),
            pl.BlockSpec((D, D * D), lambda i: (0, 0)),
            pl.BlockSpec((D * D, D), lambda i: (0, 0)),
            pl.BlockSpec((D, D), lambda i: (0, 0)),
        ],
        out_specs=pl.BlockSpec((MSG_ROWS, 128), lambda i: (i, 0)),
        out_shape=jax.ShapeDtypeStruct((EROW128, 128), jnp.float32),
    )(xjp, up, RA, RB, M, Bm)


def _gru_body(agg0_ref, agg1_ref, deg0_ref, deg1_ref, out_ref, h_ref,
              wroott_ref, bconv_ref, wih_ref, bih_ref, whh_ref, bhh_ref,
              newh_ref):
    parts = []
    for g in range(8):
        sl = slice(D * g, D * (g + 1))
        out = out_ref[:, sl]
        h = h_ref[:, sl]
        invdeg = 1.0 / jnp.maximum(deg0_ref[:, sl] + deg1_ref[:, sl], 1.0)
        agg = (agg0_ref[:, sl] + agg1_ref[:, sl]) * invdeg
        m = jnp.maximum(
            agg + jnp.dot(out, wroott_ref[...],
                          preferred_element_type=jnp.float32)
            + bconv_ref[...], 0.0)
        gi = (jnp.dot(m, wih_ref[...], preferred_element_type=jnp.float32)
              + bih_ref[...])
        gh = (jnp.dot(h, whh_ref[...], preferred_element_type=jnp.float32)
              + bhh_ref[...])
        r = jax.nn.sigmoid(gi[:, :D] + gh[:, :D])
        z = jax.nn.sigmoid(gi[:, D:2 * D] + gh[:, D:2 * D])
        n = jnp.tanh(gi[:, 2 * D:] + r * gh[:, 2 * D:])
        parts.append((1.0 - z) * n + z * h)
    newh_ref[...] = jnp.concatenate(parts, axis=1)


def _gru(agg0, agg1, deg0, deg1, outp, hp, WrootT, bconv, WihT, bih, WhhT,
         bhh):
    return pl.pallas_call(
        _gru_body,
        out_shape=jax.ShapeDtypeStruct((NROW128, 128), jnp.float32),
    )(agg0, agg1, deg0, deg1, outp, hp, WrootT, bconv, WihT, bih, WhhT, bhh)


def _set2set_body(out_ref, wihs_ref, bihs_ref, whhs_ref, bhhs_ref,
                  wihm_ref, bm_ref, hx_ref, cx_ref):
    out = out_ref[...]
    q_star = jnp.zeros((1, 2 * D), jnp.float32)
    hs = jnp.zeros((1, D), jnp.float32)
    cs = jnp.zeros((1, D), jnp.float32)
    for _ in range(6):
        g = (jnp.dot(q_star, wihs_ref[...], preferred_element_type=jnp.float32)
             + bihs_ref[...]
             + jnp.dot(hs, whhs_ref[...], preferred_element_type=jnp.float32)
             + bhhs_ref[...])
        ig = jax.nn.sigmoid(g[:, :D])
        fg = jax.nn.sigmoid(g[:, D:2 * D])
        cg = jnp.tanh(g[:, 2 * D:3 * D])
        og = jax.nn.sigmoid(g[:, 3 * D:])
        cs = fg * cs + ig * cg
        hs = og * jnp.tanh(cs)
        e = jnp.sum(out * hs, axis=1, keepdims=True)
        emax = jnp.max(e)
        a = jnp.exp(e - emax)
        asum = jnp.sum(a)
        rvec = jnp.sum(a * out, axis=0, keepdims=True) / asum
        q_star = jnp.concatenate([hs, rvec], axis=1)
    g = (jnp.dot(q_star, wihm_ref[...], preferred_element_type=jnp.float32)
         + bm_ref[...])
    ig = jax.nn.sigmoid(g[:, :D])
    fg = jax.nn.sigmoid(g[:, D:2 * D])
    cg = jnp.tanh(g[:, 2 * D:3 * D])
    og = jax.nn.sigmoid(g[:, 3 * D:])
    cx = ig * cg
    hx_ref[...] = og * jnp.tanh(cx)
    cx_ref[...] = cx


def _set2set(out, Wih_s, bih_s, Whh_s, bhh_s, Wih_m, bih_m, bhh_m):
    return pl.pallas_call(
        _set2set_body,
        out_shape=(jax.ShapeDtypeStruct((1, D), jnp.float32),
                   jax.ShapeDtypeStruct((1, D), jnp.float32)),
    )(out, Wih_s.T, bih_s.reshape(1, -1), Whh_s.T, bhh_s.reshape(1, -1),
      Wih_m.T, (bih_m + bhh_m).reshape(1, -1))


def _mlp_body(sel_ref, repcol_ref, w1at_ref, w1bsum_ref, b1_ref, w2t_ref,
              b2_ref, o_ref):
    z2 = jnp.maximum(
        jnp.dot(sel_ref[...], w1at_ref[...], preferred_element_type=jnp.float32)
        + repcol_ref[...] * w1bsum_ref[...]
        + b1_ref[...], 0.0)
    o_ref[...] = (jnp.dot(z2, w2t_ref[...], preferred_element_type=jnp.float32)
                  + b2_ref[...])


def _mlp(sel, repcol, W1, b1, W2, b2):
    n_t = sel.shape[0]
    return pl.pallas_call(
        _mlp_body,
        out_shape=jax.ShapeDtypeStruct((n_t, W2.shape[0]), jnp.float32),
    )(sel, repcol, W1[:, :4 * D].T, W1[:, 4 * D:].sum(axis=1).reshape(1, D),
      b1.reshape(1, -1), W2.T, b2.reshape(1, -1))


def kernel(x, edge_attr, edge_index, batch, nonring, W0, b0, We1, be1, We2,
           be2, Wroot, bconv, Wih, Whh, bih, bhh, Wih_s, Whh_s, bih_s, bhh_s,
           Wih_m, Whh_m, bih_m, bhh_m, W1, b1, W2, b2):
    src = edge_index[0]
    dst = edge_index[1]

    M = We2.reshape(D, D, D).transpose(0, 2, 1).reshape(D * D, D)
    Bm = be2.reshape(D, D)
    ii = jnp.arange(D * D) // D
    kk = jnp.arange(D * D) % D
    RA = (jnp.arange(D)[:, None] == ii[None, :]).astype(jnp.float32)
    RB = (jnp.arange(D)[:, None] == kk[None, :]).astype(jnp.float32)
    w1row = We1.T
    be1r = be1.reshape(1, D)

    n_fill = E_PAD - N_EDGES
    fill = jnp.arange(n_fill, dtype=jnp.int32)
    src_pad = jnp.concatenate([src, (fill * 521) % N_NODES])
    dst_pad = jnp.concatenate([dst, N_NODES + (fill % (N_PAD - N_NODES))])
    src2d = src_pad.reshape(E_PAD // CHUNK, CHUNK)
    dst2d = dst_pad.reshape(E_PAD // CHUNK, CHUNK)
    ea_pad = jnp.concatenate(
        [edge_attr, jnp.zeros((n_fill, 1), jnp.float32)])
    zeros_nd = jnp.zeros((N_PAD, D), jnp.float32)
    xp = jnp.concatenate([x, jnp.zeros((N_PAD - N_NODES, x.shape[1]),
                                       jnp.float32)])

    out16 = _node_embed(xp, W0, b0)
    outp = out16.reshape(NROW128, 128)
    hp = outp

    u16 = _u_table(ea_pad, w1row, be1r)
    up = u16.reshape(EROW128, 128)

    ones_nd = jnp.ones((E_PAD, D), jnp.float32)
    degp = _sc_scatter_add(ones_nd, dst2d, zeros_nd)
    degpk = degp.reshape(NC, NROW128, 128)
    deg0 = degpk[0]
    deg1 = degpk[1]

    WrootT = Wroot.T
    bconvr = bconv.reshape(1, D)
    WihT = Wih.T
    bihr = bih.reshape(1, 3 * D)
    WhhT = Whh.T
    bhhr = bhh.reshape(1, 3 * D)

    for _ in range(6):
        xj = _sc_gather(outp.reshape(N_PAD, D), src2d)
        msgp = _messages(xj.reshape(EROW128, 128), up, RA, RB, M, Bm)
        aggp = _sc_scatter_add(msgp.reshape(E_PAD, D), dst2d, zeros_nd)
        aggpk = aggp.reshape(NC, NROW128, 128)
        hp = _gru(aggpk[0], aggpk[1], deg0, deg1, outp, hp,
                  WrootT, bconvr, WihT, bihr, WhhT, bhhr)
        outp = hp

    out = outp.reshape(N_PAD, D)
    outv = out[:N_NODES]
    hx, cx = _set2set(outv, Wih_s, bih_s, Whh_s, bhh_s, Wih_m, bih_m, bhh_m)

    sel_rows = jnp.take(outv, nonring.reshape(-1), axis=0)
    sel = sel_rows.reshape(4 * D, -1).T
    n_t = sel.shape[0]
    repcol = jnp.repeat(hx.reshape(-1), n_t // D).reshape(n_t, 1)
    logits = _mlp(sel, repcol, W1, b1, W2, b2)
    return logits, hx, cx


# speedup vs baseline: 5.2194x; 1.1544x over previous
import functools

import jax
import jax.numpy as jnp
from jax import lax
from jax.experimental import pallas as pl
from jax.experimental.pallas import tpu as pltpu
from jax.experimental.pallas import tpu_sc as plsc

D = 16
N_NODES = 10000
N_EDGES = 160000

NC = 2
NS = 16
NW = NC * NS
CHUNK = 128
EPW = 5120
NCHUNK = EPW // CHUNK
E_PAD = NW * EPW
N_PAD = 10240
NROW128 = N_PAD * D // 128
TPT = NROW128 // NS
EROW128 = E_PAD * D // 128
EPT = EROW128 // NW
EDGE_BLOCK = 4096

_SC_MESH = plsc.VectorSubcoreMesh(core_axis_name="c", subcore_axis_name="s")


def _gather_body(tab_hbm, src2d_hbm, out_hbm, idx_v, rows_v, tab_sh, sem):
    c = lax.axis_index("c")
    s = lax.axis_index("s")
    wid = s * NC + c
    base = wid * NCHUNK
    npt = N_PAD // NS
    pltpu.sync_copy(tab_hbm.at[pl.ds(s * npt, npt)],
                    tab_sh.at[pl.ds(s * npt, npt)])
    pltpu.sync_copy(src2d_hbm.at[pl.ds(base, NCHUNK)], idx_v)
    plsc.subcore_barrier()

    def fire(j, carry):
        pltpu.async_copy(tab_sh.at[idx_v.at[j]],
                         rows_v.at[pl.ds(j * CHUNK, CHUNK)], sem)
        return carry

    lax.fori_loop(0, NCHUNK, fire, 0)

    def drain(j, carry):
        pltpu.make_async_copy(tab_sh.at[idx_v.at[j]],
                              rows_v.at[pl.ds(j * CHUNK, CHUNK)],
                              sem).wait()
        return carry

    lax.fori_loop(0, NCHUNK, drain, 0)
    pltpu.sync_copy(rows_v, out_hbm.at[pl.ds(wid * EPW, EPW)])


def _sc_gather(tab, src2d):
    return pl.kernel(
        _gather_body,
        out_type=jax.ShapeDtypeStruct((E_PAD, D), jnp.float32),
        mesh=_SC_MESH,
        compiler_params=pltpu.CompilerParams(use_tc_tiling_on_sc=False),
        scratch_types=[
            pltpu.VMEM((NCHUNK, CHUNK), jnp.int32),
            pltpu.VMEM((EPW, D), jnp.float32),
            pltpu.VMEM_SHARED((N_PAD, D), jnp.float32),
            pltpu.SemaphoreType.DMA,
        ],
    )(tab, src2d)


def _scatter_body(msg_hbm, dst2d_hbm, zeros_hbm, out_hbm, idx_v, rows_v,
                  obuf_v, agg_sh, sem):
    c = lax.axis_index("c")
    s = lax.axis_index("s")
    wid = s * NC + c
    base = wid * NCHUNK
    npt = N_PAD // NS
    pltpu.sync_copy(dst2d_hbm.at[pl.ds(base, NCHUNK)], idx_v)
    pltpu.async_copy(msg_hbm.at[pl.ds(wid * EPW, EPW)], rows_v, sem)
    pltpu.sync_copy(zeros_hbm.at[pl.ds(s * npt, npt)],
                    agg_sh.at[pl.ds(s * npt, npt)])
    pltpu.make_async_copy(msg_hbm.at[pl.ds(wid * EPW, EPW)], rows_v,
                          sem).wait()
    plsc.subcore_barrier()

    def body(j, carry):
        pltpu.sync_copy(rows_v.at[pl.ds(j * CHUNK, CHUNK)],
                        agg_sh.at[idx_v.at[j]], add=True)
        return carry

    lax.fori_loop(0, NCHUNK, body, 0)
    plsc.subcore_barrier()
    pltpu.sync_copy(agg_sh.at[pl.ds(s * npt, npt)], obuf_v)
    pltpu.sync_copy(obuf_v, out_hbm.at[c, pl.ds(s * npt, npt)])


def _sc_scatter_add(msg, dst2d, zeros_nd):
    return pl.kernel(
        _scatter_body,
        out_type=jax.ShapeDtypeStruct((NC, N_PAD, D), jnp.float32),
        mesh=_SC_MESH,
        compiler_params=pltpu.CompilerParams(use_tc_tiling_on_sc=False),
        scratch_types=[
            pltpu.VMEM((NCHUNK, CHUNK), jnp.int32),
            pltpu.VMEM((EPW, D), jnp.float32),
            pltpu.VMEM((N_PAD // NS, D), jnp.float32),
            pltpu.VMEM_SHARED((N_PAD, D), jnp.float32),
            pltpu.SemaphoreType.DMA,
        ],
    )(msg, dst2d, zeros_nd)


def _embed_body(x24_ref, wp_ref, b0t_ref, o_ref):
    o_ref[...] = jnp.maximum(
        jnp.dot(x24_ref[...], wp_ref[...], preferred_element_type=jnp.float32)
        + b0t_ref[...], 0.0)


def _node_embed(x24, WP, b0t):
    return pl.pallas_call(
        _embed_body,
        out_shape=jax.ShapeDtypeStruct((NROW128, 128), jnp.float32),
    )(x24, WP, b0t)


def _u_body(ea8_ref, re_ref, w1t_ref, be1t_ref, o_ref):
    ap = jnp.dot(ea8_ref[...], re_ref[...], preferred_element_type=jnp.float32)
    o_ref[...] = jnp.maximum(ap * w1t_ref[...] + be1t_ref[...], 0.0)


def _u_table(ea8, RE, w1t, be1t):
    grid = 10
    return pl.pallas_call(
        _u_body,
        grid=(grid,),
        in_specs=[
            pl.BlockSpec((EROW128 // grid, 8), lambda i: (i, 0)),
            pl.BlockSpec((8, 128), lambda i: (0, 0)),
            pl.BlockSpec((1, 128), lambda i: (0, 0)),
            pl.BlockSpec((1, 128), lambda i: (0, 0)),
        ],
        out_specs=pl.BlockSpec((EROW128 // grid, 128), lambda i: (i, 0)),
        out_shape=jax.ShapeDtypeStruct((EROW128, 128), jnp.float32),
    )(ea8, RE, w1t, be1t)


MSG_ROWS = 2048


def _msg_body(xp_ref, up_ref, RA_ref, RB_ref, M_ref, Bm_ref, o_ref):
    X = xp_ref[...]
    U = up_ref[...]
    RA = RA_ref[...]
    RB = RB_ref[...]
    M = M_ref[...]
    Bm = Bm_ref[...]
    parts = []
    for g in range(8):
        xg = X[:, D * g:D * (g + 1)]
        ug = U[:, D * g:D * (g + 1)]
        A = jnp.dot(xg, RA, preferred_element_type=jnp.float32)
        B = jnp.dot(ug, RB, preferred_element_type=jnp.float32)
        parts.append(
            jnp.dot(A * B, M, preferred_element_type=jnp.float32)
            + jnp.dot(xg, Bm, preferred_element_type=jnp.float32))
    o_ref[...] = jnp.concatenate(parts, axis=1)


def _messages(xjp, up, RA, RB, M, Bm):
    grid = EROW128 // MSG_ROWS
    return pl.pallas_call(
        _msg_body,
        grid=(grid,),
        in_specs=[
            pl.BlockSpec((MSG_ROWS, 128), lambda i: (i, 0)),
            pl.BlockSpec((MSG_ROWS, 128), lambda i: (i, 0)),
            pl.BlockSpec((D, D * D), lambda i: (0, 0)),
            pl.BlockSpec((D, D * D), lambda i: (0, 0)),
            pl.BlockSpec((D * D, D), lambda i: (0, 0)),
            pl.BlockSpec((D, D), lambda i: (0, 0)),
        ],
        out_specs=pl.BlockSpec((MSG_ROWS, 128), lambda i: (i, 0)),
        out_shape=jax.ShapeDtypeStruct((EROW128, 128), jnp.float32),
    )(xjp, up, RA, RB, M, Bm)


def _gru_body(agg0_ref, agg1_ref, deg0_ref, deg1_ref, out_ref, h_ref,
              wroott_ref, bconv_ref, wih_ref, bih_ref, whh_ref, bhh_ref,
              newh_ref):
    parts = []
    for g in range(8):
        sl = slice(D * g, D * (g + 1))
        out = out_ref[:, sl]
        h = h_ref[:, sl]
        invdeg = 1.0 / jnp.maximum(deg0_ref[:, sl] + deg1_ref[:, sl], 1.0)
        agg = (agg0_ref[:, sl] + agg1_ref[:, sl]) * invdeg
        m = jnp.maximum(
            agg + jnp.dot(out, wroott_ref[...],
                          preferred_element_type=jnp.float32)
            + bconv_ref[...], 0.0)
        gi = (jnp.dot(m, wih_ref[...], preferred_element_type=jnp.float32)
              + bih_ref[...])
        gh = (jnp.dot(h, whh_ref[...], preferred_element_type=jnp.float32)
              + bhh_ref[...])
        r = jax.nn.sigmoid(gi[:, :D] + gh[:, :D])
        z = jax.nn.sigmoid(gi[:, D:2 * D] + gh[:, D:2 * D])
        n = jnp.tanh(gi[:, 2 * D:] + r * gh[:, 2 * D:])
        parts.append((1.0 - z) * n + z * h)
    newh_ref[...] = jnp.concatenate(parts, axis=1)


def _gru(agg0, agg1, deg0, deg1, outp, hp, WrootT, bconv, WihT, bih, WhhT,
         bhh):
    return pl.pallas_call(
        _gru_body,
        out_shape=jax.ShapeDtypeStruct((NROW128, 128), jnp.float32),
    )(agg0, agg1, deg0, deg1, outp, hp, WrootT, bconv, WihT, bih, WhhT, bhh)


def _set2set_body(out_ref, wihs_ref, bihs_ref, whhs_ref, bhhs_ref,
                  wihm_ref, bm_ref, hx_ref, cx_ref):
    out = out_ref[...]
    q_star = jnp.zeros((1, 2 * D), jnp.float32)
    hs = jnp.zeros((1, D), jnp.float32)
    cs = jnp.zeros((1, D), jnp.float32)
    for _ in range(6):
        g = (jnp.dot(q_star, wihs_ref[...], preferred_element_type=jnp.float32)
             + bihs_ref[...]
             + jnp.dot(hs, whhs_ref[...], preferred_element_type=jnp.float32)
             + bhhs_ref[...])
        ig = jax.nn.sigmoid(g[:, :D])
        fg = jax.nn.sigmoid(g[:, D:2 * D])
        cg = jnp.tanh(g[:, 2 * D:3 * D])
        og = jax.nn.sigmoid(g[:, 3 * D:])
        cs = fg * cs + ig * cg
        hs = og * jnp.tanh(cs)
        e = jnp.sum(out * hs, axis=1, keepdims=True)
        emax = jnp.max(e)
        a = jnp.exp(e - emax)
        asum = jnp.sum(a)
        rvec = jnp.sum(a * out, axis=0, keepdims=True) / asum
        q_star = jnp.concatenate([hs, rvec], axis=1)
    g = (jnp.dot(q_star, wihm_ref[...], preferred_element_type=jnp.float32)
         + bm_ref[...])
    ig = jax.nn.sigmoid(g[:, :D])
    fg = jax.nn.sigmoid(g[:, D:2 * D])
    cg = jnp.tanh(g[:, 2 * D:3 * D])
    og = jax.nn.sigmoid(g[:, 3 * D:])
    cx = ig * cg
    hx_ref[...] = og * jnp.tanh(cx)
    cx_ref[...] = cx


def _set2set(out, Wih_s, bih_s, Whh_s, bhh_s, Wih_m, bih_m, bhh_m):
    return pl.pallas_call(
        _set2set_body,
        out_shape=(jax.ShapeDtypeStruct((1, D), jnp.float32),
                   jax.ShapeDtypeStruct((1, D), jnp.float32)),
    )(out, Wih_s.T, bih_s.reshape(1, -1), Whh_s.T, bhh_s.reshape(1, -1),
      Wih_m.T, (bih_m + bhh_m).reshape(1, -1))


def _mlp_body(sel_ref, repcol_ref, w1at_ref, w1bsum_ref, b1_ref, w2t_ref,
              b2_ref, o_ref):
    z2 = jnp.maximum(
        jnp.dot(sel_ref[...], w1at_ref[...], preferred_element_type=jnp.float32)
        + repcol_ref[...] * w1bsum_ref[...]
        + b1_ref[...], 0.0)
    o_ref[...] = (jnp.dot(z2, w2t_ref[...], preferred_element_type=jnp.float32)
                  + b2_ref[...])


def _mlp(sel, repcol, W1, b1, W2, b2):
    n_t = sel.shape[0]
    return pl.pallas_call(
        _mlp_body,
        out_shape=jax.ShapeDtypeStruct((n_t, W2.shape[0]), jnp.float32),
    )(sel, repcol, W1[:, :4 * D].T, W1[:, 4 * D:].sum(axis=1).reshape(1, D),
      b1.reshape(1, -1), W2.T, b2.reshape(1, -1))


def kernel(x, edge_attr, edge_index, batch, nonring, W0, b0, We1, be1, We2,
           be2, Wroot, bconv, Wih, Whh, bih, bhh, Wih_s, Whh_s, bih_s, bhh_s,
           Wih_m, Whh_m, bih_m, bhh_m, W1, b1, W2, b2):
    src = edge_index[0]
    dst = edge_index[1]

    M = We2.reshape(D, D, D).transpose(0, 2, 1).reshape(D * D, D)
    Bm = be2.reshape(D, D)
    ii = jnp.arange(D * D) // D
    kk = jnp.arange(D * D) % D
    RA = (jnp.arange(D)[:, None] == ii[None, :]).astype(jnp.float32)
    RB = (jnp.arange(D)[:, None] == kk[None, :]).astype(jnp.float32)
    w1row = We1.T
    be1r = be1.reshape(1, D)

    n_fill = E_PAD - N_EDGES
    fill = jnp.arange(n_fill, dtype=jnp.int32)
    src_pad = jnp.concatenate([src, (fill * 521) % N_NODES])
    dst_pad = jnp.concatenate([dst, N_NODES + (fill % (N_PAD - N_NODES))])
    src2d = src_pad.reshape(E_PAD // CHUNK, CHUNK)
    dst2d = dst_pad.reshape(E_PAD // CHUNK, CHUNK)
    ea8 = jnp.concatenate(
        [edge_attr.reshape(-1), jnp.zeros((n_fill,), jnp.float32)]
    ).reshape(EROW128, 8)
    zeros_nd = jnp.zeros((N_PAD, D), jnp.float32)
    x24 = jnp.concatenate(
        [x.reshape(-1),
         jnp.zeros(((N_PAD - N_NODES) * x.shape[1],), jnp.float32)]
    ).reshape(NROW128, 3 * 8)

    gidx = jnp.arange(128) // D
    kidx = jnp.arange(128) % D
    WP = jnp.zeros((24, 128), jnp.float32)
    WP = WP.at[3 * gidx + 0, jnp.arange(128)].set(W0[kidx, 0])
    WP = WP.at[3 * gidx + 1, jnp.arange(128)].set(W0[kidx, 1])
    WP = WP.at[3 * gidx + 2, jnp.arange(128)].set(W0[kidx, 2])
    b0t = jnp.tile(b0, 8).reshape(1, 128)
    RE = (jnp.arange(8)[:, None] == gidx[None, :]).astype(jnp.float32)
    w1t = jnp.tile(We1.reshape(-1), 8).reshape(1, 128)
    be1t = jnp.tile(be1, 8).reshape(1, 128)

    outp = _node_embed(x24, WP, b0t)
    hp = outp

    up = _u_table(ea8, RE, w1t, be1t)

    ones_nd = jnp.ones((E_PAD, D), jnp.float32)
    degp = _sc_scatter_add(ones_nd, dst2d, zeros_nd)
    degpk = degp.reshape(NC, NROW128, 128)
    deg0 = degpk[0]
    deg1 = degpk[1]

    WrootT = Wroot.T
    bconvr = bconv.reshape(1, D)
    WihT = Wih.T
    bihr = bih.reshape(1, 3 * D)
    WhhT = Whh.T
    bhhr = bhh.reshape(1, 3 * D)

    for _ in range(6):
        xj = _sc_gather(outp.reshape(N_PAD, D), src2d)
        msgp = _messages(xj.reshape(EROW128, 128), up, RA, RB, M, Bm)
        aggp = _sc_scatter_add(msgp.reshape(E_PAD, D), dst2d, zeros_nd)
        aggpk = aggp.reshape(NC, NROW128, 128)
        hp = _gru(aggpk[0], aggpk[1], deg0, deg1, outp, hp,
                  WrootT, bconvr, WihT, bihr, WhhT, bhhr)
        outp = hp

    out = outp.reshape(N_PAD, D)
    outv = out[:N_NODES]
    hx, cx = _set2set(outv, Wih_s, bih_s, Whh_s, bhh_s, Wih_m, bih_m, bhh_m)

    sel_rows = jnp.take(outv, nonring.reshape(-1), axis=0)
    sel = sel_rows.reshape(4 * D, -1).T
    n_t = sel.shape[0]
    repcol = jnp.repeat(hx.reshape(-1), n_t // D).reshape(n_t, 1)
    logits = _mlp(sel, repcol, W1, b1, W2, b2)
    return logits, hx, cx
